# baseline (speedup 1.0000x reference)
import functools

import jax
import jax.numpy as jnp
from jax import lax
from jax.experimental import pallas as pl
from jax.experimental.pallas import tpu as pltpu
from jax.experimental.pallas import tpu_sc as plsc

_DIM = 32
_VPAD = 128
_NC = 2
_NS = 16
_NW = _NC * _NS
_BT = 512


@functools.cache
def _make_sc_kernel(npos, nbatch):
    steps = npos
    mesh = plsc.VectorSubcoreMesh(core_axis_name="c", subcore_axis_name="s")

    @functools.partial(
        pl.kernel,
        mesh=mesh,
        compiler_params=pltpu.CompilerParams(
            needs_layout_passes=False, use_tc_tiling_on_sc=True),
        out_type=jax.ShapeDtypeStruct((npos, _DIM, nbatch), jnp.float32),
        scratch_types=[
            pltpu.VMEM((_VPAD // 4, 128), jnp.float32),
            pltpu.VMEM((_DIM * _DIM // 128, 128), jnp.float32),
            pltpu.VMEM((128,), jnp.float32),
            pltpu.VMEM((_VPAD // 4, 128), jnp.float32),
            pltpu.VMEM((_DIM, _VPAD), jnp.float32),
            pltpu.VMEM((2, _BT), jnp.int32),
            pltpu.VMEM((2, _DIM, _BT), jnp.float32),
            pltpu.SemaphoreType.DMA,
            pltpu.SemaphoreType.DMA,
        ],
    )
    def sc_kernel(idx_hbm, e_hbm, w_hbm, b_hbm, out_hbm,
                  e_v, w_v, b_v, t_f, t_t, idx_v, st, sem_i, sem_o):
        pltpu.sync_copy(e_hbm, e_v)
        pltpu.sync_copy(w_hbm, w_v)
        pltpu.sync_copy(b_hbm, b_v)
        b0 = b_v[pl.ds(0, 16)]
        b1 = b_v[pl.ds(16, 16)]

        def build_row(v, carry):
            r = v >> 2
            c = (v & 3) * 32
            rowv = jnp.full((16,), r, dtype=jnp.int32)
            acc0, acc1 = b0, b1
            for k in range(_DIM):
                ek = plsc.load_gather(
                    e_v, [rowv, jnp.full((16,), c + k, dtype=jnp.int32)])
                wf = k * _DIM
                w0 = w_v[wf // 128, pl.ds(wf % 128, 16)]
                w1 = w_v[wf // 128, pl.ds(wf % 128 + 16, 16)]
                acc0 = acc0 + ek * w0
                acc1 = acc1 + ek * w1
            t_f[r, pl.ds(c, 16)] = acc0
            t_f[r, pl.ds(c + 16, 16)] = acc1
            return carry

        lax.fori_loop(0, _VPAD, build_row, 0)

        lane = lax.iota(jnp.int32, 16)
        for j in range(_DIM):
            for vb in range(_VPAD // 16):
                a = (lane + vb * 16) * _DIM + j
                g = plsc.load_gather(t_f, [a >> 7, a & 127])
                t_t[j, pl.ds(vb * 16, 16)] = g

        wid = lax.axis_index("s") * _NC + lax.axis_index("c")
        col0 = wid * _BT

        def fire_idx(l, p):
            pltpu.async_copy(
                idx_hbm.at[l, pl.ds(col0, _BT)], idx_v.at[p], sem_i)

        def wait_idx(p):
            pltpu.make_async_copy(
                idx_hbm.at[0, pl.ds(col0, _BT)], idx_v.at[p], sem_i).wait()

        def wait_out():
            pltpu.make_async_copy(
                st.at[0], out_hbm.at[0, :, pl.ds(col0, _BT)], sem_o).wait()

        fire_idx(0, 0)

        def outer(o, carry):
            for p in range(2):
                l = o * 2 + p
                wait_idx(p)

                @pl.when(l + 1 < steps)
                def _prefetch():
                    fire_idx(l + 1, 1 - p)

                @pl.when(l >= 2)
                def _reclaim():
                    wait_out()

                sp = st.at[p]

                @plsc.parallel_loop(0, _BT // 16, unroll=4)
                def _gather(bb):
                    b0 = bb * 16
                    iv = idx_v[p, pl.ds(b0, 16)]
                    for j in range(_DIM):
                        g = plsc.load_gather(
                            t_t, [jnp.full((16,), j, dtype=jnp.int32), iv])
                        sp[j, pl.ds(b0, 16)] = g

                pltpu.async_copy(
                    sp, out_hbm.at[l, :, pl.ds(col0, _BT)], sem_o)
            return carry

        lax.fori_loop(0, steps // 2, outer, 0)
        wait_out()
        wait_out()

    return sc_kernel


def kernel(x, embed_table, W, b):
    bsz, hist = x.shape
    xt = x.T.astype(jnp.int32)
    e2 = jnp.pad(embed_table.astype(jnp.float32),
                 ((0, _VPAD - embed_table.shape[0]), (0, 0))).reshape(-1, 128)
    w2 = W.astype(jnp.float32).T.reshape(-1, 128)
    b2 = jnp.pad(b.astype(jnp.float32), (0, 128 - _DIM))
    out3 = _make_sc_kernel(hist, bsz)(xt, e2, w2, b2)
    return out3.transpose(2, 0, 1)


# speedup vs baseline: 1.5457x; 1.5457x over previous
import functools

import jax
import jax.numpy as jnp
from jax import lax
from jax.experimental import pallas as pl
from jax.experimental.pallas import tpu as pltpu
from jax.experimental.pallas import tpu_sc as plsc

_DIM = 32
_VPAD = 128
_NC = 2
_NS = 16
_NW = _NC * _NS
_BT = 512


@functools.cache
def _make_sc_kernel(npos, nbatch):
    steps = npos
    mesh = plsc.VectorSubcoreMesh(core_axis_name="c", subcore_axis_name="s")

    @functools.partial(
        pl.kernel,
        mesh=mesh,
        compiler_params=pltpu.CompilerParams(
            needs_layout_passes=False, use_tc_tiling_on_sc=True),
        out_type=jax.ShapeDtypeStruct((npos, _DIM, nbatch), jnp.float32),
        scratch_types=[
            pltpu.VMEM((_VPAD // 4, 128), jnp.float32),
            pltpu.VMEM((_DIM * _DIM // 128, 128), jnp.float32),
            pltpu.VMEM((128,), jnp.float32),
            pltpu.VMEM((_VPAD // 4, 128), jnp.float32),
            pltpu.VMEM((_DIM * _VPAD * 16,), jnp.float32),
            pltpu.VMEM((2, _BT), jnp.int32),
            pltpu.VMEM((2, _DIM, _BT), jnp.float32),
            pltpu.SemaphoreType.DMA,
            pltpu.SemaphoreType.DMA,
        ],
    )
    def sc_kernel(idx_hbm, e_hbm, w_hbm, b_hbm, out_hbm,
                  e_v, w_v, b_v, t_f, t_rep, idx_v, st, sem_i, sem_o):
        pltpu.sync_copy(e_hbm, e_v)
        pltpu.sync_copy(w_hbm, w_v)
        pltpu.sync_copy(b_hbm, b_v)
        b0 = b_v[pl.ds(0, 16)]
        b1 = b_v[pl.ds(16, 16)]

        def build_row(v, carry):
            r = v >> 2
            c = (v & 3) * 32
            rowv = jnp.full((16,), r, dtype=jnp.int32)
            acc0, acc1 = b0, b1
            for k in range(_DIM):
                ek = plsc.load_gather(
                    e_v, [rowv, jnp.full((16,), c + k, dtype=jnp.int32)])
                wf = k * _DIM
                w0 = w_v[wf // 128, pl.ds(wf % 128, 16)]
                w1 = w_v[wf // 128, pl.ds(wf % 128 + 16, 16)]
                acc0 = acc0 + ek * w0
                acc1 = acc1 + ek * w1
            t_f[r, pl.ds(c, 16)] = acc0
            t_f[r, pl.ds(c + 16, 16)] = acc1
            return carry

        lax.fori_loop(0, _VPAD, build_row, 0)

        lane = lax.iota(jnp.int32, 16)
        lsel = [jnp.full((16,), i, dtype=jnp.int32) for i in range(16)]

        def rep_row(v, carry):
            r = v >> 2
            c = (v & 3) * 32
            r0 = t_f[r, pl.ds(c, 16)]
            r1 = t_f[r, pl.ds(c + 16, 16)]
            for j in range(_DIM):
                src = r0 if j < 16 else r1
                bv = lax.gather(
                    src, lsel[j % 16][:, None],
                    lax.GatherDimensionNumbers(
                        offset_dims=(), collapsed_slice_dims=(0,),
                        start_index_map=(0,)),
                    (1,), mode=lax.GatherScatterMode.PROMISE_IN_BOUNDS)
                t_rep[pl.ds(v * 16 + j * (_VPAD * 16), 16)] = bv
            return carry

        lax.fori_loop(0, _VPAD, rep_row, 0)

        wid = lax.axis_index("s") * _NC + lax.axis_index("c")
        col0 = wid * _BT

        def fire_idx(l, p):
            pltpu.async_copy(
                idx_hbm.at[l, pl.ds(col0, _BT)], idx_v.at[p], sem_i)

        def wait_idx(p):
            pltpu.make_async_copy(
                idx_hbm.at[0, pl.ds(col0, _BT)], idx_v.at[p], sem_i).wait()

        def wait_out():
            pltpu.make_async_copy(
                st.at[0], out_hbm.at[0, :, pl.ds(col0, _BT)], sem_o).wait()

        fire_idx(0, 0)

        def outer(o, carry):
            for p in range(2):
                l = o * 2 + p
                wait_idx(p)

                @pl.when(l + 1 < steps)
                def _prefetch():
                    fire_idx(l + 1, 1 - p)

                @pl.when(l >= 2)
                def _reclaim():
                    wait_out()

                sp = st.at[p]

                @plsc.parallel_loop(0, _BT // 16, unroll=2)
                def _gather(bb):
                    b0 = bb * 16
                    iv = idx_v[p, pl.ds(b0, 16)]
                    ivb = (iv << 4) + lane
                    for j in range(_DIM):
                        g = plsc.load_gather(t_rep, [ivb + j * (_VPAD * 16)])
                        sp[j, pl.ds(b0, 16)] = g

                pltpu.async_copy(
                    sp, out_hbm.at[l, :, pl.ds(col0, _BT)], sem_o)
            return carry

        lax.fori_loop(0, steps // 2, outer, 0)
        wait_out()
        wait_out()

    return sc_kernel


def kernel(x, embed_table, W, b):
    bsz, hist = x.shape
    xt = x.T.astype(jnp.int32)
    e2 = jnp.pad(embed_table.astype(jnp.float32),
                 ((0, _VPAD - embed_table.shape[0]), (0, 0))).reshape(-1, 128)
    w2 = W.astype(jnp.float32).T.reshape(-1, 128)
    b2 = jnp.pad(b.astype(jnp.float32), (0, 128 - _DIM))
    out3 = _make_sc_kernel(hist, bsz)(xt, e2, w2, b2)
    return out3.transpose(2, 0, 1)
